# Initial kernel scaffold; baseline (speedup 1.0000x reference)
import jax
import jax.numpy as jnp
from jax.experimental import pallas as pl


def kernel(points, spoints):
    raise NotImplementedError("write your pallas kernel here")


# speedup vs baseline: 6.4387x; 6.4387x over previous
import functools

import jax
import jax.numpy as jnp
from jax import lax
from jax.experimental import pallas as pl

_K = 16
_ROWS = 256


def _voro_body(points_ref, spointst_ref, out_ref, *, m_total):
    p = points_ref[0]
    s = spointst_ref[0]

    px = p[:, 0:1]
    py = p[:, 1:2]
    pz = p[:, 2:3]
    sx = s[0:1, :]
    sy = s[1:2, :]
    sz = s[2:3, :]

    d = (px - sx) ** 2 + (py - sy) ** 2 + (pz - sz) ** 2

    col = lax.broadcasted_iota(jnp.int32, d.shape, 1)
    inf = jnp.float32(jnp.inf)

    d0 = None
    cx = cy = cz = None
    best = None
    for k in range(_K):
        m = jnp.min(d, axis=1, keepdims=True)
        hit = d <= m
        a = jnp.min(jnp.where(hit, col, m_total), axis=1, keepdims=True)
        sel = col == a
        gx = jnp.sum(jnp.where(sel, sx, 0.0), axis=1, keepdims=True)
        gy = jnp.sum(jnp.where(sel, sy, 0.0), axis=1, keepdims=True)
        gz = jnp.sum(jnp.where(sel, sz, 0.0), axis=1, keepdims=True)
        if k == 0:
            d0, cx, cy, cz = m, gx, gy, gz
        else:
            el2 = (gx - cx) ** 2 + (gy - cy) ** 2 + (gz - cz) ** 2
            val = (m - d0) ** 2 / (4.0 * el2)
            best = val if best is None else jnp.minimum(best, val)
        if k != _K - 1:
            d = jnp.where(sel, inf, d)

    out_ref[0, 0, :] = best[:, 0]


def kernel(points, spoints):
    b, n, _ = points.shape
    m = spoints.shape[1]
    spoints_t = jnp.swapaxes(spoints, 1, 2)

    nblk = n // _ROWS
    grid = (b, nblk)
    out = pl.pallas_call(
        functools.partial(_voro_body, m_total=m),
        grid=grid,
        in_specs=[
            pl.BlockSpec((1, _ROWS, 3), lambda bi, i: (bi, i, 0)),
            pl.BlockSpec((1, 3, m), lambda bi, i: (bi, 0, 0)),
        ],
        out_specs=pl.BlockSpec((1, 1, _ROWS), lambda bi, i: (bi * nblk + i, 0, 0)),
        out_shape=jax.ShapeDtypeStruct((b * nblk, 1, _ROWS), jnp.float32),
    )(points, spoints_t)
    return out.reshape(b, n)


# speedup vs baseline: 16.6651x; 2.5883x over previous
import functools

import jax
import jax.numpy as jnp
from jax import lax
from jax.experimental import pallas as pl
from jax.experimental.pallas import tpu as pltpu
from jax.experimental.pallas import tpu_sc as plsc

_K = 16
_ROWS = 256
_NC = 2
_NS = 16
_NW = _NC * _NS


def _topk_body(points_ref, spointst_ref, vals_ref, idx_ref, *, m_total):
    p = points_ref[0]
    s = spointst_ref[0]

    px = p[:, 0:1]
    py = p[:, 1:2]
    pz = p[:, 2:3]
    sx = s[0:1, :]
    sy = s[1:2, :]
    sz = s[2:3, :]

    d = (px - sx) ** 2 + (py - sy) ** 2 + (pz - sz) ** 2

    col = lax.broadcasted_iota(jnp.int32, d.shape, 1)
    inf = jnp.float32(jnp.inf)

    ms, asel = [], []
    for k in range(_K):
        m = jnp.min(d, axis=1, keepdims=True)
        a = jnp.min(jnp.where(d == m, col, m_total), axis=1, keepdims=True)
        ms.append(m)
        asel.append(a)
        if k != _K - 1:
            d = jnp.where(col == a, inf, d)

    vals_ref[0] = jnp.concatenate(ms, axis=1)
    idx_ref[0] = jnp.concatenate(asel, axis=1)


def _topk_call(points, spoints_t):
    b, n, _ = points.shape
    m = spoints_t.shape[2]
    nblk = n // _ROWS
    vals, idx = pl.pallas_call(
        functools.partial(_topk_body, m_total=m),
        grid=(b, nblk),
        in_specs=[
            pl.BlockSpec((1, _ROWS, 3), lambda bi, i: (bi, i, 0)),
            pl.BlockSpec((1, 3, m), lambda bi, i: (bi, 0, 0)),
        ],
        out_specs=[
            pl.BlockSpec((1, _ROWS, _K), lambda bi, i: (bi * nblk + i, 0, 0)),
            pl.BlockSpec((1, _ROWS, _K), lambda bi, i: (bi * nblk + i, 0, 0)),
        ],
        out_shape=[
            jax.ShapeDtypeStruct((b * nblk, _ROWS, _K), jnp.float32),
            jax.ShapeDtypeStruct((b * nblk, _ROWS, _K), jnp.int32),
        ],
    )(points, spoints_t)
    return vals.reshape(b, n, _K), idx.reshape(b, n, _K)


def _voro_sc(spoints_t, vals, idx):
    b, n, _ = vals.shape
    m = spoints_t.shape[2]
    rw = (b * n) // _NW
    ng = rw // 16
    mesh = plsc.VectorSubcoreMesh(core_axis_name="c", subcore_axis_name="s")

    spt_flat = spoints_t.reshape(b, 3 * m)
    vals_flat = vals.reshape(b, n * _K)
    idx_flat = idx.reshape(b, n * _K)

    @functools.partial(
        pl.kernel,
        mesh=mesh,
        out_type=jax.ShapeDtypeStruct((b, n), jnp.float32),
        scratch_types=[
            pltpu.VMEM((3 * m,), jnp.float32),
            pltpu.VMEM((rw * _K,), jnp.float32),
            pltpu.VMEM((rw * _K,), jnp.int32),
            pltpu.VMEM((rw,), jnp.float32),
        ],
        compiler_params=pltpu.CompilerParams(needs_layout_passes=False),
    )
    def sc_kernel(spt_hbm, vals_hbm, idx_hbm, out_hbm, spt_v, vals_v, idx_v, best_v):
        wid = lax.axis_index("s") * _NC + lax.axis_index("c")
        base = wid * rw
        bb = base // n
        start = base % n
        pltpu.sync_copy(spt_hbm.at[bb], spt_v)
        pltpu.sync_copy(vals_hbm.at[bb, pl.ds(start * _K, rw * _K)], vals_v)
        pltpu.sync_copy(idx_hbm.at[bb, pl.ds(start * _K, rw * _K)], idx_v)

        def group(g, carry):
            flat0 = (lax.iota(jnp.int32, 16) + g * 16) * _K
            i0 = plsc.load_gather(idx_v, [flat0])
            d0 = plsc.load_gather(vals_v, [flat0])
            cx = plsc.load_gather(spt_v, [i0])
            cy = plsc.load_gather(spt_v, [i0 + m])
            cz = plsc.load_gather(spt_v, [i0 + 2 * m])
            best = jnp.full((16,), jnp.inf, jnp.float32)
            for k in range(1, _K):
                ik = plsc.load_gather(idx_v, [flat0 + k])
                dk = plsc.load_gather(vals_v, [flat0 + k])
                gx = plsc.load_gather(spt_v, [ik])
                gy = plsc.load_gather(spt_v, [ik + m])
                gz = plsc.load_gather(spt_v, [ik + 2 * m])
                ex = gx - cx
                ey = gy - cy
                ez = gz - cz
                el2 = ex * ex + ey * ey + ez * ez
                diff = dk - d0
                best = jnp.minimum(best, diff * diff / (4.0 * el2))
            best_v[pl.ds(g * 16, 16)] = best
            return carry

        lax.fori_loop(0, ng, group, 0)
        pltpu.sync_copy(best_v, out_hbm.at[bb, pl.ds(start, rw)])

    return sc_kernel(spt_flat, vals_flat, idx_flat)


def kernel(points, spoints):
    spoints_t = jnp.swapaxes(spoints, 1, 2)
    vals, idx = _topk_call(points, spoints_t)
    return _voro_sc(spoints_t, vals, idx)


# speedup vs baseline: 35.9875x; 2.1595x over previous
import functools

import jax
import jax.numpy as jnp
from jax import lax
from jax.experimental import pallas as pl
from jax.experimental.pallas import tpu as pltpu
from jax.experimental.pallas import tpu_sc as plsc

_K = 16
_ROWS = 256
_IDXBITS = 12
_NC = 2
_NS = 16
_NW = _NC * _NS


def _topk_body(points_ref, spointst_ref, keys_ref, *, m_total):
    p = points_ref[0]
    s = spointst_ref[0]

    px = p[:, 0:1]
    py = p[:, 1:2]
    pz = p[:, 2:3]
    sx = s[0:1, :]
    sy = s[1:2, :]
    sz = s[2:3, :]

    d = (px - sx) ** 2 + (py - sy) ** 2 + (pz - sz) ** 2

    col = lax.broadcasted_iota(jnp.int32, d.shape, 1)
    mask = jnp.int32(-(1 << _IDXBITS))
    kbits = (lax.bitcast_convert_type(d, jnp.int32) & mask) | col
    keys = lax.bitcast_convert_type(kbits, jnp.float32)
    inf = jnp.float32(jnp.inf)

    outs = []
    for k in range(_K):
        kmin = jnp.min(keys, axis=1, keepdims=True)
        outs.append(kmin)
        if k != _K - 1:
            keys = jnp.where(keys == kmin, inf, keys)

    keys_ref[0] = lax.bitcast_convert_type(
        jnp.concatenate(outs, axis=1), jnp.int32)


def _topk_call(points, spoints_t):
    b, n, _ = points.shape
    m = spoints_t.shape[2]
    nblk = n // _ROWS
    keys = pl.pallas_call(
        functools.partial(_topk_body, m_total=m),
        grid=(b, nblk),
        in_specs=[
            pl.BlockSpec((1, _ROWS, 3), lambda bi, i: (bi, i, 0)),
            pl.BlockSpec((1, 3, m), lambda bi, i: (bi, 0, 0)),
        ],
        out_specs=pl.BlockSpec((1, _ROWS, _K), lambda bi, i: (bi * nblk + i, 0, 0)),
        out_shape=jax.ShapeDtypeStruct((b * nblk, _ROWS, _K), jnp.int32),
    )(points, spoints_t)
    return keys.reshape(b, n * _K)


def _voro_sc(points_t, spoints_t, keys_flat):
    b = points_t.shape[0]
    n = points_t.shape[2]
    m = spoints_t.shape[2]
    rw = (b * n) // _NW
    ng = rw // 16
    mesh = plsc.VectorSubcoreMesh(core_axis_name="c", subcore_axis_name="s")

    pts_flat = points_t.reshape(b, 3 * n)
    spt_flat = spoints_t.reshape(b, 3 * m)
    imask = jnp.int32((1 << _IDXBITS) - 1)

    @functools.partial(
        pl.kernel,
        mesh=mesh,
        out_type=jax.ShapeDtypeStruct((b, n), jnp.float32),
        scratch_types=[
            pltpu.VMEM((3 * m,), jnp.float32),
            pltpu.VMEM((rw * _K,), jnp.int32),
            pltpu.VMEM((rw,), jnp.float32),
            pltpu.VMEM((rw,), jnp.float32),
            pltpu.VMEM((rw,), jnp.float32),
            pltpu.VMEM((_K * 16,), jnp.float32),
            pltpu.VMEM((_K * 16,), jnp.int32),
            pltpu.VMEM((rw,), jnp.float32),
        ],
        compiler_params=pltpu.CompilerParams(needs_layout_passes=False),
    )
    def sc_kernel(pts_hbm, spt_hbm, keys_hbm, out_hbm,
                  spt_v, keys_v, px_v, py_v, pz_v, dk2_s, ik_s, best_v):
        wid = lax.axis_index("s") * _NC + lax.axis_index("c")
        base = wid * rw
        bb = base // n
        start = base % n
        pltpu.sync_copy(spt_hbm.at[bb], spt_v)
        pltpu.sync_copy(keys_hbm.at[bb, pl.ds(start * _K, rw * _K)], keys_v)
        pltpu.sync_copy(pts_hbm.at[bb, pl.ds(start, rw)], px_v)
        pltpu.sync_copy(pts_hbm.at[bb, pl.ds(n + start, rw)], py_v)
        pltpu.sync_copy(pts_hbm.at[bb, pl.ds(2 * n + start, rw)], pz_v)

        inf16 = jnp.full((16,), jnp.inf, jnp.float32)

        def group(g, carry):
            off = g * 16
            flat0 = (lax.iota(jnp.int32, 16) + off) * _K
            px = px_v[pl.ds(off, 16)]
            py = py_v[pl.ds(off, 16)]
            pz = pz_v[pl.ds(off, 16)]

            best0 = inf16
            cx = jnp.zeros((16,), jnp.float32)
            cy = jnp.zeros((16,), jnp.float32)
            cz = jnp.zeros((16,), jnp.float32)
            i0 = jnp.zeros((16,), jnp.int32)
            for k in range(_K):
                key = plsc.load_gather(keys_v, [flat0 + k])
                ik = key & imask
                gx = plsc.load_gather(spt_v, [ik])
                gy = plsc.load_gather(spt_v, [ik + m])
                gz = plsc.load_gather(spt_v, [ik + 2 * m])
                dx = px - gx
                dy = py - gy
                dz = pz - gz
                dk2 = dx * dx + dy * dy + dz * dz
                ik_s[pl.ds(k * 16, 16)] = ik
                dk2_s[pl.ds(k * 16, 16)] = dk2
                upd = dk2 < best0
                cx = jnp.where(upd, gx, cx)
                cy = jnp.where(upd, gy, cy)
                cz = jnp.where(upd, gz, cz)
                i0 = jnp.where(upd, ik, i0)
                best0 = jnp.where(upd, dk2, best0)

            best = inf16
            for k in range(_K):
                ik = ik_s[pl.ds(k * 16, 16)]
                dk2 = dk2_s[pl.ds(k * 16, 16)]
                gx = plsc.load_gather(spt_v, [ik])
                gy = plsc.load_gather(spt_v, [ik + m])
                gz = plsc.load_gather(spt_v, [ik + 2 * m])
                ex = gx - cx
                ey = gy - cy
                ez = gz - cz
                el2 = ex * ex + ey * ey + ez * ez
                num = dk2 - best0
                val = num * num / (4.0 * el2)
                val = jnp.where(ik == i0, jnp.inf, val)
                best = jnp.minimum(best, val)
            best_v[pl.ds(off, 16)] = best
            return carry

        lax.fori_loop(0, ng, group, 0)
        pltpu.sync_copy(best_v, out_hbm.at[bb, pl.ds(start, rw)])

    return sc_kernel(pts_flat, spt_flat, keys_flat)


def kernel(points, spoints):
    points_t = jnp.swapaxes(points, 1, 2)
    spoints_t = jnp.swapaxes(spoints, 1, 2)
    keys = _topk_call(points, spoints_t)
    return _voro_sc(points_t, spoints_t, keys)


# speedup vs baseline: 39.2061x; 1.0894x over previous
import functools

import jax
import jax.numpy as jnp
from jax import lax
from jax.experimental import pallas as pl
from jax.experimental.pallas import tpu as pltpu
from jax.experimental.pallas import tpu_sc as plsc

_K = 16
_ROWS = 256
_IDXBITS = 12
_NC = 2
_NS = 16
_NW = _NC * _NS


def _topk_body(points_ref, spointst_ref, keys_ref, *, m_total):
    p = points_ref[0]
    s = spointst_ref[0]

    sx = s[0:1, :]
    sy = s[1:2, :]
    sz = s[2:3, :]

    s2 = sx * sx + sy * sy + sz * sz
    pd = jax.lax.dot_general(p, s, (((1,), (0,)), ((), ())),
                             preferred_element_type=jnp.float32)
    d = s2 - (pd + pd)

    col = lax.broadcasted_iota(jnp.int32, d.shape, 1)
    mask = jnp.int32(-(1 << _IDXBITS))
    kbits = (lax.bitcast_convert_type(d, jnp.int32) & mask) | col
    keys = lax.bitcast_convert_type(kbits, jnp.float32)
    inf = jnp.float32(jnp.inf)

    outs = []
    for k in range(_K):
        kmin = jnp.min(keys, axis=1, keepdims=True)
        outs.append(kmin)
        if k != _K - 1:
            keys = jnp.where(keys == kmin, inf, keys)

    keys_ref[0] = lax.bitcast_convert_type(
        jnp.concatenate(outs, axis=1), jnp.int32)


def _topk_call(points, spoints_t):
    b, n, _ = points.shape
    m = spoints_t.shape[2]
    nblk = n // _ROWS
    keys = pl.pallas_call(
        functools.partial(_topk_body, m_total=m),
        grid=(b, nblk),
        in_specs=[
            pl.BlockSpec((1, _ROWS, 3), lambda bi, i: (bi, i, 0)),
            pl.BlockSpec((1, 3, m), lambda bi, i: (bi, 0, 0)),
        ],
        out_specs=pl.BlockSpec((1, _ROWS, _K), lambda bi, i: (bi * nblk + i, 0, 0)),
        out_shape=jax.ShapeDtypeStruct((b * nblk, _ROWS, _K), jnp.int32),
    )(points, spoints_t)
    return keys.reshape(b, n * _K)


def _voro_sc(points_t, spoints_t, keys_flat):
    b = points_t.shape[0]
    n = points_t.shape[2]
    m = spoints_t.shape[2]
    rw = (b * n) // _NW
    ng = rw // 16
    mesh = plsc.VectorSubcoreMesh(core_axis_name="c", subcore_axis_name="s")

    pts_flat = points_t.reshape(b, 3 * n)
    spt_flat = spoints_t.reshape(b, 3 * m)
    imask = jnp.int32((1 << _IDXBITS) - 1)

    @functools.partial(
        pl.kernel,
        mesh=mesh,
        out_type=jax.ShapeDtypeStruct((b, n), jnp.float32),
        scratch_types=[
            pltpu.VMEM((3 * m,), jnp.float32),
            pltpu.VMEM((rw * _K,), jnp.int32),
            pltpu.VMEM((rw,), jnp.float32),
            pltpu.VMEM((rw,), jnp.float32),
            pltpu.VMEM((rw,), jnp.float32),
            pltpu.VMEM((_K * 16,), jnp.float32),
            pltpu.VMEM((_K * 16,), jnp.int32),
            pltpu.VMEM((rw,), jnp.float32),
        ],
        compiler_params=pltpu.CompilerParams(needs_layout_passes=False),
    )
    def sc_kernel(pts_hbm, spt_hbm, keys_hbm, out_hbm,
                  spt_v, keys_v, px_v, py_v, pz_v, dk2_s, ik_s, best_v):
        wid = lax.axis_index("s") * _NC + lax.axis_index("c")
        base = wid * rw
        bb = base // n
        start = base % n
        pltpu.sync_copy(spt_hbm.at[bb], spt_v)
        pltpu.sync_copy(keys_hbm.at[bb, pl.ds(start * _K, rw * _K)], keys_v)
        pltpu.sync_copy(pts_hbm.at[bb, pl.ds(start, rw)], px_v)
        pltpu.sync_copy(pts_hbm.at[bb, pl.ds(n + start, rw)], py_v)
        pltpu.sync_copy(pts_hbm.at[bb, pl.ds(2 * n + start, rw)], pz_v)

        inf16 = jnp.full((16,), jnp.inf, jnp.float32)

        def group(g, carry):
            off = g * 16
            flat0 = (lax.iota(jnp.int32, 16) + off) * _K
            px = px_v[pl.ds(off, 16)]
            py = py_v[pl.ds(off, 16)]
            pz = pz_v[pl.ds(off, 16)]

            best0 = inf16
            cx = jnp.zeros((16,), jnp.float32)
            cy = jnp.zeros((16,), jnp.float32)
            cz = jnp.zeros((16,), jnp.float32)
            i0 = jnp.zeros((16,), jnp.int32)
            for k in range(_K):
                key = plsc.load_gather(keys_v, [flat0 + k])
                ik = key & imask
                gx = plsc.load_gather(spt_v, [ik])
                gy = plsc.load_gather(spt_v, [ik + m])
                gz = plsc.load_gather(spt_v, [ik + 2 * m])
                dx = px - gx
                dy = py - gy
                dz = pz - gz
                dk2 = dx * dx + dy * dy + dz * dz
                ik_s[pl.ds(k * 16, 16)] = ik
                dk2_s[pl.ds(k * 16, 16)] = dk2
                upd = dk2 < best0
                cx = jnp.where(upd, gx, cx)
                cy = jnp.where(upd, gy, cy)
                cz = jnp.where(upd, gz, cz)
                i0 = jnp.where(upd, ik, i0)
                best0 = jnp.where(upd, dk2, best0)

            best = inf16
            for k in range(_K):
                ik = ik_s[pl.ds(k * 16, 16)]
                dk2 = dk2_s[pl.ds(k * 16, 16)]
                gx = plsc.load_gather(spt_v, [ik])
                gy = plsc.load_gather(spt_v, [ik + m])
                gz = plsc.load_gather(spt_v, [ik + 2 * m])
                ex = gx - cx
                ey = gy - cy
                ez = gz - cz
                el2 = ex * ex + ey * ey + ez * ez
                num = dk2 - best0
                val = num * num / (4.0 * el2)
                val = jnp.where(ik == i0, jnp.inf, val)
                best = jnp.minimum(best, val)
            best_v[pl.ds(off, 16)] = best
            return carry

        lax.fori_loop(0, ng, group, 0)
        pltpu.sync_copy(best_v, out_hbm.at[bb, pl.ds(start, rw)])

    return sc_kernel(pts_flat, spt_flat, keys_flat)


def kernel(points, spoints):
    points_t = jnp.swapaxes(points, 1, 2)
    spoints_t = jnp.swapaxes(spoints, 1, 2)
    keys = _topk_call(points, spoints_t)
    return _voro_sc(points_t, spoints_t, keys)


# speedup vs baseline: 39.5893x; 1.0098x over previous
import functools

import jax
import jax.numpy as jnp
from jax import lax
from jax.experimental import pallas as pl
from jax.experimental.pallas import tpu as pltpu
from jax.experimental.pallas import tpu_sc as plsc

_K = 16
_ROWS = 512
_IDXBITS = 12
_NC = 2
_NS = 16
_NW = _NC * _NS


def _topk_body(points_ref, spointst_ref, keys_ref, *, m_total):
    p = points_ref[0]
    s = spointst_ref[0]

    sx = s[0:1, :]
    sy = s[1:2, :]
    sz = s[2:3, :]

    s2 = sx * sx + sy * sy + sz * sz
    pd = jax.lax.dot_general(p, s, (((1,), (0,)), ((), ())),
                             preferred_element_type=jnp.float32)
    d = s2 - (pd + pd)

    col = lax.broadcasted_iota(jnp.int32, d.shape, 1)
    mask = jnp.int32(-(1 << _IDXBITS))
    kbits = (lax.bitcast_convert_type(d, jnp.int32) & mask) | col
    keys = lax.bitcast_convert_type(kbits, jnp.float32)
    inf = jnp.float32(jnp.inf)

    outs = []
    for k in range(_K):
        kmin = jnp.min(keys, axis=1, keepdims=True)
        outs.append(kmin)
        if k != _K - 1:
            keys = jnp.where(keys == kmin, inf, keys)

    keys_ref[0] = lax.bitcast_convert_type(
        jnp.concatenate(outs, axis=1), jnp.int32)


def _topk_call(points, spoints_t):
    b, n, _ = points.shape
    m = spoints_t.shape[2]
    nblk = n // _ROWS
    keys = pl.pallas_call(
        functools.partial(_topk_body, m_total=m),
        grid=(b, nblk),
        in_specs=[
            pl.BlockSpec((1, _ROWS, 3), lambda bi, i: (bi, i, 0)),
            pl.BlockSpec((1, 3, m), lambda bi, i: (bi, 0, 0)),
        ],
        out_specs=pl.BlockSpec((1, _ROWS, _K), lambda bi, i: (bi * nblk + i, 0, 0)),
        out_shape=jax.ShapeDtypeStruct((b * nblk, _ROWS, _K), jnp.int32),
    )(points, spoints_t)
    return keys.reshape(b, n * _K)


def _voro_sc(points_t, spoints_t, keys_flat):
    b = points_t.shape[0]
    n = points_t.shape[2]
    m = spoints_t.shape[2]
    rw = (b * n) // _NW
    ng = rw // 16
    mesh = plsc.VectorSubcoreMesh(core_axis_name="c", subcore_axis_name="s")

    pts_flat = points_t.reshape(b, 3 * n)
    spt_flat = spoints_t.reshape(b, 3 * m)
    imask = jnp.int32((1 << _IDXBITS) - 1)

    @functools.partial(
        pl.kernel,
        mesh=mesh,
        out_type=jax.ShapeDtypeStruct((b, n), jnp.float32),
        scratch_types=[
            pltpu.VMEM((3 * m,), jnp.float32),
            pltpu.VMEM((rw * _K,), jnp.int32),
            pltpu.VMEM((rw,), jnp.float32),
            pltpu.VMEM((rw,), jnp.float32),
            pltpu.VMEM((rw,), jnp.float32),
            pltpu.VMEM((_K * 16,), jnp.float32),
            pltpu.VMEM((_K * 16,), jnp.int32),
            pltpu.VMEM((rw,), jnp.float32),
        ],
        compiler_params=pltpu.CompilerParams(needs_layout_passes=False),
    )
    def sc_kernel(pts_hbm, spt_hbm, keys_hbm, out_hbm,
                  spt_v, keys_v, px_v, py_v, pz_v, dk2_s, ik_s, best_v):
        wid = lax.axis_index("s") * _NC + lax.axis_index("c")
        base = wid * rw
        bb = base // n
        start = base % n
        pltpu.sync_copy(spt_hbm.at[bb], spt_v)
        pltpu.sync_copy(keys_hbm.at[bb, pl.ds(start * _K, rw * _K)], keys_v)
        pltpu.sync_copy(pts_hbm.at[bb, pl.ds(start, rw)], px_v)
        pltpu.sync_copy(pts_hbm.at[bb, pl.ds(n + start, rw)], py_v)
        pltpu.sync_copy(pts_hbm.at[bb, pl.ds(2 * n + start, rw)], pz_v)

        inf16 = jnp.full((16,), jnp.inf, jnp.float32)

        def group(g, carry):
            off = g * 16
            flat0 = (lax.iota(jnp.int32, 16) + off) * _K
            px = px_v[pl.ds(off, 16)]
            py = py_v[pl.ds(off, 16)]
            pz = pz_v[pl.ds(off, 16)]

            best0 = inf16
            cx = jnp.zeros((16,), jnp.float32)
            cy = jnp.zeros((16,), jnp.float32)
            cz = jnp.zeros((16,), jnp.float32)
            i0 = jnp.zeros((16,), jnp.int32)
            for k in range(_K):
                key = plsc.load_gather(keys_v, [flat0 + k])
                ik = key & imask
                gx = plsc.load_gather(spt_v, [ik])
                gy = plsc.load_gather(spt_v, [ik + m])
                gz = plsc.load_gather(spt_v, [ik + 2 * m])
                dx = px - gx
                dy = py - gy
                dz = pz - gz
                dk2 = dx * dx + dy * dy + dz * dz
                ik_s[pl.ds(k * 16, 16)] = ik
                dk2_s[pl.ds(k * 16, 16)] = dk2
                upd = dk2 < best0
                cx = jnp.where(upd, gx, cx)
                cy = jnp.where(upd, gy, cy)
                cz = jnp.where(upd, gz, cz)
                i0 = jnp.where(upd, ik, i0)
                best0 = jnp.where(upd, dk2, best0)

            best = inf16
            for k in range(_K):
                ik = ik_s[pl.ds(k * 16, 16)]
                dk2 = dk2_s[pl.ds(k * 16, 16)]
                gx = plsc.load_gather(spt_v, [ik])
                gy = plsc.load_gather(spt_v, [ik + m])
                gz = plsc.load_gather(spt_v, [ik + 2 * m])
                ex = gx - cx
                ey = gy - cy
                ez = gz - cz
                el2 = ex * ex + ey * ey + ez * ez
                num = dk2 - best0
                val = num * num / (4.0 * el2)
                val = jnp.where(ik == i0, jnp.inf, val)
                best = jnp.minimum(best, val)
            best_v[pl.ds(off, 16)] = best
            return carry

        lax.fori_loop(0, ng, group, 0)
        pltpu.sync_copy(best_v, out_hbm.at[bb, pl.ds(start, rw)])

    return sc_kernel(pts_flat, spt_flat, keys_flat)


def kernel(points, spoints):
    points_t = jnp.swapaxes(points, 1, 2)
    spoints_t = jnp.swapaxes(spoints, 1, 2)
    keys = _topk_call(points, spoints_t)
    return _voro_sc(points_t, spoints_t, keys)


# speedup vs baseline: 41.7427x; 1.0544x over previous
import functools

import jax
import jax.numpy as jnp
from jax import lax
from jax.experimental import pallas as pl
from jax.experimental.pallas import tpu as pltpu
from jax.experimental.pallas import tpu_sc as plsc

_K = 16
_ROWS = 512
_IDXBITS = 12
_NC = 2
_NS = 16
_NW = _NC * _NS


def _topk_body(points_ref, spointst_ref, keys_ref, *, m_total):
    p = points_ref[0]
    s = spointst_ref[0]

    sx = s[0:1, :]
    sy = s[1:2, :]
    sz = s[2:3, :]

    s2 = sx * sx + sy * sy + sz * sz
    pd = jax.lax.dot_general(p, s, (((1,), (0,)), ((), ())),
                             preferred_element_type=jnp.float32)
    d = s2 - (pd + pd)

    col = lax.broadcasted_iota(jnp.int32, d.shape, 1)
    mask = jnp.int32(-(1 << _IDXBITS))
    kbits = (lax.bitcast_convert_type(d, jnp.int32) & mask) | col
    keys = lax.bitcast_convert_type(kbits, jnp.float32)
    inf = jnp.float32(jnp.inf)

    q = m_total // 4
    a0 = keys[:, 0:q]
    a1 = keys[:, q:2 * q]
    a2 = keys[:, 2 * q:3 * q]
    a3 = keys[:, 3 * q:4 * q]
    lo01 = jnp.minimum(a0, a1)
    hi01 = jnp.maximum(a0, a1)
    lo23 = jnp.minimum(a2, a3)
    hi23 = jnp.maximum(a2, a3)
    t0 = jnp.minimum(lo01, lo23)
    x = jnp.maximum(lo01, lo23)
    y = jnp.minimum(hi01, hi23)
    t3 = jnp.maximum(hi01, hi23)
    t1 = jnp.minimum(x, y)
    t2 = jnp.maximum(x, y)

    outs = []
    for k in range(_K):
        kmin = jnp.min(t0, axis=1, keepdims=True)
        outs.append(kmin)
        if k != _K - 1:
            sel = t0 == kmin
            t0 = jnp.where(sel, t1, t0)
            t1 = jnp.where(sel, t2, t1)
            t2 = jnp.where(sel, t3, t2)
            t3 = jnp.where(sel, inf, t3)

    keys_ref[0] = lax.bitcast_convert_type(
        jnp.concatenate(outs, axis=1), jnp.int32)


def _topk_call(points, spoints_t):
    b, n, _ = points.shape
    m = spoints_t.shape[2]
    nblk = n // _ROWS
    keys = pl.pallas_call(
        functools.partial(_topk_body, m_total=m),
        grid=(b, nblk),
        in_specs=[
            pl.BlockSpec((1, _ROWS, 3), lambda bi, i: (bi, i, 0)),
            pl.BlockSpec((1, 3, m), lambda bi, i: (bi, 0, 0)),
        ],
        out_specs=pl.BlockSpec((1, _ROWS, _K), lambda bi, i: (bi * nblk + i, 0, 0)),
        out_shape=jax.ShapeDtypeStruct((b * nblk, _ROWS, _K), jnp.int32),
    )(points, spoints_t)
    return keys.reshape(b, n * _K)


def _voro_sc(points_t, spoints_t, keys_flat):
    b = points_t.shape[0]
    n = points_t.shape[2]
    m = spoints_t.shape[2]
    rw = (b * n) // _NW
    ng = rw // 16
    mesh = plsc.VectorSubcoreMesh(core_axis_name="c", subcore_axis_name="s")

    pts_flat = points_t.reshape(b, 3 * n)
    spt_flat = spoints_t.reshape(b, 3 * m)
    imask = jnp.int32((1 << _IDXBITS) - 1)

    @functools.partial(
        pl.kernel,
        mesh=mesh,
        out_type=jax.ShapeDtypeStruct((b, n), jnp.float32),
        scratch_types=[
            pltpu.VMEM((3 * m,), jnp.float32),
            pltpu.VMEM((rw * _K,), jnp.int32),
            pltpu.VMEM((rw,), jnp.float32),
            pltpu.VMEM((rw,), jnp.float32),
            pltpu.VMEM((rw,), jnp.float32),
            pltpu.VMEM((_K * 16,), jnp.float32),
            pltpu.VMEM((_K * 16,), jnp.int32),
            pltpu.VMEM((rw,), jnp.float32),
        ],
        compiler_params=pltpu.CompilerParams(needs_layout_passes=False),
    )
    def sc_kernel(pts_hbm, spt_hbm, keys_hbm, out_hbm,
                  spt_v, keys_v, px_v, py_v, pz_v, dk2_s, ik_s, best_v):
        wid = lax.axis_index("s") * _NC + lax.axis_index("c")
        base = wid * rw
        bb = base // n
        start = base % n
        pltpu.sync_copy(spt_hbm.at[bb], spt_v)
        pltpu.sync_copy(keys_hbm.at[bb, pl.ds(start * _K, rw * _K)], keys_v)
        pltpu.sync_copy(pts_hbm.at[bb, pl.ds(start, rw)], px_v)
        pltpu.sync_copy(pts_hbm.at[bb, pl.ds(n + start, rw)], py_v)
        pltpu.sync_copy(pts_hbm.at[bb, pl.ds(2 * n + start, rw)], pz_v)

        inf16 = jnp.full((16,), jnp.inf, jnp.float32)

        def group(g, carry):
            off = g * 16
            flat0 = (lax.iota(jnp.int32, 16) + off) * _K
            px = px_v[pl.ds(off, 16)]
            py = py_v[pl.ds(off, 16)]
            pz = pz_v[pl.ds(off, 16)]

            best0 = inf16
            cx = jnp.zeros((16,), jnp.float32)
            cy = jnp.zeros((16,), jnp.float32)
            cz = jnp.zeros((16,), jnp.float32)
            i0 = jnp.zeros((16,), jnp.int32)
            for k in range(_K):
                key = plsc.load_gather(keys_v, [flat0 + k])
                ik = key & imask
                gx = plsc.load_gather(spt_v, [ik])
                gy = plsc.load_gather(spt_v, [ik + m])
                gz = plsc.load_gather(spt_v, [ik + 2 * m])
                dx = px - gx
                dy = py - gy
                dz = pz - gz
                dk2 = dx * dx + dy * dy + dz * dz
                ik_s[pl.ds(k * 16, 16)] = ik
                dk2_s[pl.ds(k * 16, 16)] = dk2
                upd = dk2 < best0
                cx = jnp.where(upd, gx, cx)
                cy = jnp.where(upd, gy, cy)
                cz = jnp.where(upd, gz, cz)
                i0 = jnp.where(upd, ik, i0)
                best0 = jnp.where(upd, dk2, best0)

            best = inf16
            for k in range(_K):
                ik = ik_s[pl.ds(k * 16, 16)]
                dk2 = dk2_s[pl.ds(k * 16, 16)]
                gx = plsc.load_gather(spt_v, [ik])
                gy = plsc.load_gather(spt_v, [ik + m])
                gz = plsc.load_gather(spt_v, [ik + 2 * m])
                ex = gx - cx
                ey = gy - cy
                ez = gz - cz
                el2 = ex * ex + ey * ey + ez * ez
                num = dk2 - best0
                val = num * num / (4.0 * el2)
                val = jnp.where(ik == i0, jnp.inf, val)
                best = jnp.minimum(best, val)
            best_v[pl.ds(off, 16)] = best
            return carry

        lax.fori_loop(0, ng, group, 0)
        pltpu.sync_copy(best_v, out_hbm.at[bb, pl.ds(start, rw)])

    return sc_kernel(pts_flat, spt_flat, keys_flat)


def kernel(points, spoints):
    points_t = jnp.swapaxes(points, 1, 2)
    spoints_t = jnp.swapaxes(spoints, 1, 2)
    keys = _topk_call(points, spoints_t)
    return _voro_sc(points_t, spoints_t, keys)


# speedup vs baseline: 71.2091x; 1.7059x over previous
import functools

import jax
import jax.numpy as jnp
from jax import lax
from jax.experimental import pallas as pl
from jax.experimental.pallas import tpu as pltpu
from jax.experimental.pallas import tpu_sc as plsc

_K = 16
_ROWS = 512
_IDXBITS = 12
_FOLD = 4
_NC = 2
_NS = 16
_NW = _NC * _NS


def _topk_body(points_ref, spointst_ref, keys_ref, *, m_total):
    p = points_ref[0]
    s = spointst_ref[0]

    sx = s[0:1, :]
    sy = s[1:2, :]
    sz = s[2:3, :]

    s2 = sx * sx + sy * sy + sz * sz
    pd = jax.lax.dot_general(p, s, (((1,), (0,)), ((), ())),
                             preferred_element_type=jnp.float32)
    d = s2 - (pd + pd)

    col = lax.broadcasted_iota(jnp.int32, d.shape, 1)
    mask = jnp.int32(-(1 << _IDXBITS))
    kbits = (lax.bitcast_convert_type(d, jnp.int32) & mask) | col
    keys = lax.bitcast_convert_type(kbits, jnp.float32)
    inf = jnp.float32(jnp.inf)

    q = m_total // _FOLD
    t0 = keys[:, 0:q]
    for g in range(1, _FOLD):
        t0 = jnp.minimum(t0, keys[:, g * q:(g + 1) * q])

    outs = []
    for k in range(_K):
        kmin = jnp.min(t0, axis=1, keepdims=True)
        outs.append(kmin)
        if k != _K - 1:
            t0 = jnp.where(t0 == kmin, inf, t0)

    keys_ref[0] = lax.bitcast_convert_type(
        jnp.concatenate(outs, axis=1), jnp.int32)


def _topk_call(points, spoints_t):
    b, n, _ = points.shape
    m = spoints_t.shape[2]
    nblk = n // _ROWS
    keys = pl.pallas_call(
        functools.partial(_topk_body, m_total=m),
        grid=(b, nblk),
        in_specs=[
            pl.BlockSpec((1, _ROWS, 3), lambda bi, i: (bi, i, 0)),
            pl.BlockSpec((1, 3, m), lambda bi, i: (bi, 0, 0)),
        ],
        out_specs=pl.BlockSpec((1, _ROWS, _K), lambda bi, i: (bi * nblk + i, 0, 0)),
        out_shape=jax.ShapeDtypeStruct((b * nblk, _ROWS, _K), jnp.int32),
    )(points, spoints_t)
    return keys.reshape(b, n * _K)


def _merge_cap(a, b, cap):
    lo = [jnp.minimum(x, y) for x, y in zip(a, b[::-1])]
    hi = [jnp.maximum(x, y) for x, y in zip(a, b[::-1])]

    def clean(v):
        nv = len(v)
        if nv <= 1:
            return v
        h = nv // 2
        lo2 = [jnp.minimum(v[i], v[i + h]) for i in range(h)]
        hi2 = [jnp.maximum(v[i], v[i + h]) for i in range(h)]
        return clean(lo2) + clean(hi2)

    if 2 * len(a) > cap:
        return clean(lo)
    return clean(lo) + clean(hi)


def _voro_sc(points_t, spoints_t, keys_flat):
    b = points_t.shape[0]
    n = points_t.shape[2]
    m = spoints_t.shape[2]
    qw = m // _FOLD
    nc = _K * _FOLD
    rw = (b * n) // _NW
    ng = rw // 16
    mesh = plsc.VectorSubcoreMesh(core_axis_name="c", subcore_axis_name="s")

    pts_flat = points_t.reshape(b, 3 * n)
    spt_flat = spoints_t.reshape(b, 3 * m)
    imask = jnp.int32((1 << _IDXBITS) - 1)
    qmask = jnp.int32(qw - 1)

    @functools.partial(
        pl.kernel,
        mesh=mesh,
        out_type=jax.ShapeDtypeStruct((b, n), jnp.float32),
        scratch_types=[
            pltpu.VMEM((3 * m,), jnp.float32),
            pltpu.VMEM((rw * _K,), jnp.int32),
            pltpu.VMEM((rw,), jnp.float32),
            pltpu.VMEM((rw,), jnp.float32),
            pltpu.VMEM((rw,), jnp.float32),
            pltpu.VMEM((nc * 16,), jnp.float32),
            pltpu.VMEM((nc * 16,), jnp.int32),
            pltpu.VMEM((rw,), jnp.float32),
        ],
        compiler_params=pltpu.CompilerParams(needs_layout_passes=False),
    )
    def sc_kernel(pts_hbm, spt_hbm, keys_hbm, out_hbm,
                  spt_v, keys_v, px_v, py_v, pz_v, dk2_s, ik_s, best_v):
        wid = lax.axis_index("s") * _NC + lax.axis_index("c")
        base = wid * rw
        bb = base // n
        start = base % n
        pltpu.sync_copy(spt_hbm.at[bb], spt_v)
        pltpu.sync_copy(keys_hbm.at[bb, pl.ds(start * _K, rw * _K)], keys_v)
        pltpu.sync_copy(pts_hbm.at[bb, pl.ds(start, rw)], px_v)
        pltpu.sync_copy(pts_hbm.at[bb, pl.ds(n + start, rw)], py_v)
        pltpu.sync_copy(pts_hbm.at[bb, pl.ds(2 * n + start, rw)], pz_v)

        inf16 = jnp.full((16,), jnp.inf, jnp.float32)

        def group(g, carry):
            off = g * 16
            flat0 = (lax.iota(jnp.int32, 16) + off) * _K
            px = px_v[pl.ds(off, 16)]
            py = py_v[pl.ds(off, 16)]
            pz = pz_v[pl.ds(off, 16)]

            best0 = inf16
            cx = jnp.zeros((16,), jnp.float32)
            cy = jnp.zeros((16,), jnp.float32)
            cz = jnp.zeros((16,), jnp.float32)
            i0 = jnp.zeros((16,), jnp.int32)
            dks = []
            for j in range(_K):
                key = plsc.load_gather(keys_v, [flat0 + j])
                pos = key & qmask
                for gg in range(_FOLD):
                    ik = pos + gg * qw
                    gx = plsc.load_gather(spt_v, [ik])
                    gy = plsc.load_gather(spt_v, [ik + m])
                    gz = plsc.load_gather(spt_v, [ik + 2 * m])
                    dx = px - gx
                    dy = py - gy
                    dz = pz - gz
                    dk2 = dx * dx + dy * dy + dz * dz
                    sl = j * _FOLD + gg
                    ik_s[pl.ds(sl * 16, 16)] = ik
                    dk2_s[pl.ds(sl * 16, 16)] = dk2
                    dks.append(dk2)
                    upd = dk2 < best0
                    cx = jnp.where(upd, gx, cx)
                    cy = jnp.where(upd, gy, cy)
                    cz = jnp.where(upd, gz, cz)
                    i0 = jnp.where(upd, ik, i0)
                    best0 = jnp.where(upd, dk2, best0)

            lists = [[v] for v in dks]
            while len(lists) > 1:
                lists = [_merge_cap(lists[i], lists[i + 1], _K)
                         for i in range(0, len(lists), 2)]
            tau = lists[0][_K - 1]

            best = inf16
            for sl in range(nc):
                ik = ik_s[pl.ds(sl * 16, 16)]
                dk2 = dk2_s[pl.ds(sl * 16, 16)]
                gx = plsc.load_gather(spt_v, [ik])
                gy = plsc.load_gather(spt_v, [ik + m])
                gz = plsc.load_gather(spt_v, [ik + 2 * m])
                ex = gx - cx
                ey = gy - cy
                ez = gz - cz
                el2 = ex * ex + ey * ey + ez * ez
                num = dk2 - best0
                val = num * num / (4.0 * el2)
                keep = (dk2 <= tau) & (ik != i0)
                best = jnp.minimum(best, jnp.where(keep, val, jnp.inf))
            best_v[pl.ds(off, 16)] = best
            return carry

        lax.fori_loop(0, ng, group, 0)
        pltpu.sync_copy(best_v, out_hbm.at[bb, pl.ds(start, rw)])

    return sc_kernel(pts_flat, spt_flat, keys_flat)


def kernel(points, spoints):
    points_t = jnp.swapaxes(points, 1, 2)
    spoints_t = jnp.swapaxes(spoints, 1, 2)
    keys = _topk_call(points, spoints_t)
    return _voro_sc(points_t, spoints_t, keys)


# speedup vs baseline: 89.9676x; 1.2634x over previous
import functools

import jax
import jax.numpy as jnp
from jax import lax
from jax.experimental import pallas as pl
from jax.experimental.pallas import tpu as pltpu
from jax.experimental.pallas import tpu_sc as plsc

_K = 16
_ROWS = 512
_IDXBITS = 12
_FOLD = 4
_NC = 2
_NS = 16
_NW = _NC * _NS


def _topk_body(points_ref, spointst_ref, keys_ref, *, m_total):
    p = points_ref[0]
    s = spointst_ref[0]

    sx = s[0:1, :]
    sy = s[1:2, :]
    sz = s[2:3, :]

    s2 = sx * sx + sy * sy + sz * sz
    pd = jax.lax.dot_general(p, s, (((1,), (0,)), ((), ())),
                             preferred_element_type=jnp.float32)
    d = s2 - (pd + pd)

    col = lax.broadcasted_iota(jnp.int32, d.shape, 1)
    mask = jnp.int32(-(1 << _IDXBITS))
    kbits = (lax.bitcast_convert_type(d, jnp.int32) & mask) | col
    keys = lax.bitcast_convert_type(kbits, jnp.float32)
    inf = jnp.float32(jnp.inf)

    q = m_total // _FOLD
    t0 = keys[:, 0:q]
    for g in range(1, _FOLD):
        t0 = jnp.minimum(t0, keys[:, g * q:(g + 1) * q])

    outs = []
    for k in range(_K):
        kmin = jnp.min(t0, axis=1, keepdims=True)
        outs.append(kmin)
        if k != _K - 1:
            t0 = jnp.where(t0 == kmin, inf, t0)

    keys_ref[0] = lax.bitcast_convert_type(
        jnp.concatenate(outs, axis=1), jnp.int32)


def _topk_call(points, spoints_t):
    b, n, _ = points.shape
    m = spoints_t.shape[2]
    nblk = n // _ROWS
    keys = pl.pallas_call(
        functools.partial(_topk_body, m_total=m),
        grid=(b, nblk),
        in_specs=[
            pl.BlockSpec((1, _ROWS, 3), lambda bi, i: (bi, i, 0)),
            pl.BlockSpec((1, 3, m), lambda bi, i: (bi, 0, 0)),
        ],
        out_specs=pl.BlockSpec((1, _ROWS, _K), lambda bi, i: (bi * nblk + i, 0, 0)),
        out_shape=jax.ShapeDtypeStruct((b * nblk, _ROWS, _K), jnp.int32),
    )(points, spoints_t)
    return keys.reshape(b, n * _K)


def _merge_cap(a, b, cap):
    lo = [jnp.minimum(x, y) for x, y in zip(a, b[::-1])]
    hi = [jnp.maximum(x, y) for x, y in zip(a, b[::-1])]

    def clean(v):
        nv = len(v)
        if nv <= 1:
            return v
        h = nv // 2
        lo2 = [jnp.minimum(v[i], v[i + h]) for i in range(h)]
        hi2 = [jnp.maximum(v[i], v[i + h]) for i in range(h)]
        return clean(lo2) + clean(hi2)

    if 2 * len(a) > cap:
        return clean(lo)
    return clean(lo) + clean(hi)


def _voro_sc(points_t, spoints_t, keys_flat):
    b = points_t.shape[0]
    n = points_t.shape[2]
    m = spoints_t.shape[2]
    qw = m // _FOLD
    nc = _K * _FOLD
    rw = (b * n) // _NW
    ng = rw // 16
    mesh = plsc.VectorSubcoreMesh(core_axis_name="c", subcore_axis_name="s")

    pts_flat = points_t.reshape(b, 3 * n)
    spt_flat = spoints_t.reshape(b, 3 * m)
    imask = jnp.int32((1 << _IDXBITS) - 1)
    qmask = jnp.int32(qw - 1)

    @functools.partial(
        pl.kernel,
        mesh=mesh,
        out_type=jax.ShapeDtypeStruct((b, n), jnp.float32),
        scratch_types=[
            pltpu.VMEM((3 * m,), jnp.float32),
            pltpu.VMEM((rw * _K,), jnp.int32),
            pltpu.VMEM((rw,), jnp.float32),
            pltpu.VMEM((rw,), jnp.float32),
            pltpu.VMEM((rw,), jnp.float32),
            pltpu.VMEM((nc * 16,), jnp.float32),
            pltpu.VMEM((nc * 16,), jnp.int32),
            pltpu.VMEM((rw,), jnp.float32),
        ],
        compiler_params=pltpu.CompilerParams(needs_layout_passes=False),
    )
    def sc_kernel(pts_hbm, spt_hbm, keys_hbm, out_hbm,
                  spt_v, keys_v, px_v, py_v, pz_v, dk2_s, ik_s, best_v):
        wid = lax.axis_index("s") * _NC + lax.axis_index("c")
        base = wid * rw
        bb = base // n
        start = base % n
        pltpu.sync_copy(spt_hbm.at[bb], spt_v)
        pltpu.sync_copy(keys_hbm.at[bb, pl.ds(start * _K, rw * _K)], keys_v)
        pltpu.sync_copy(pts_hbm.at[bb, pl.ds(start, rw)], px_v)
        pltpu.sync_copy(pts_hbm.at[bb, pl.ds(n + start, rw)], py_v)
        pltpu.sync_copy(pts_hbm.at[bb, pl.ds(2 * n + start, rw)], pz_v)

        inf16 = jnp.full((16,), jnp.inf, jnp.float32)

        def group(g, carry):
            off = g * 16
            flat0 = (lax.iota(jnp.int32, 16) + off) * _K
            px = px_v[pl.ds(off, 16)]
            py = py_v[pl.ds(off, 16)]
            pz = pz_v[pl.ds(off, 16)]

            best0 = inf16
            cx = jnp.zeros((16,), jnp.float32)
            cy = jnp.zeros((16,), jnp.float32)
            cz = jnp.zeros((16,), jnp.float32)
            i0 = jnp.zeros((16,), jnp.int32)
            dks = []
            for j in range(_K):
                key = plsc.load_gather(keys_v, [flat0 + j])
                pos = key & qmask
                for gg in range(_FOLD):
                    ik = pos + gg * qw
                    gx = plsc.load_gather(spt_v, [ik])
                    gy = plsc.load_gather(spt_v, [ik + m])
                    gz = plsc.load_gather(spt_v, [ik + 2 * m])
                    dx = px - gx
                    dy = py - gy
                    dz = pz - gz
                    dk2 = dx * dx + dy * dy + dz * dz
                    sl = j * _FOLD + gg
                    ik_s[pl.ds(sl * 16, 16)] = ik
                    dk2_s[pl.ds(sl * 16, 16)] = dk2
                    dks.append(dk2)
                    upd = dk2 < best0
                    cx = jnp.where(upd, gx, cx)
                    cy = jnp.where(upd, gy, cy)
                    cz = jnp.where(upd, gz, cz)
                    i0 = jnp.where(upd, ik, i0)
                    best0 = jnp.where(upd, dk2, best0)

            lists = [[v] for v in dks]
            while len(lists) > 1:
                lists = [_merge_cap(lists[i], lists[i + 1], _K)
                         for i in range(0, len(lists), 2)]
            tau = lists[0][_K - 1]

            best = inf16
            for sl in range(nc):
                ik = ik_s[pl.ds(sl * 16, 16)]
                dk2 = dk2_s[pl.ds(sl * 16, 16)]
                gx = plsc.load_gather(spt_v, [ik])
                gy = plsc.load_gather(spt_v, [ik + m])
                gz = plsc.load_gather(spt_v, [ik + 2 * m])
                ex = gx - cx
                ey = gy - cy
                ez = gz - cz
                el2 = ex * ex + ey * ey + ez * ez
                num = dk2 - best0
                val = num * num / (4.0 * el2)
                keep = (dk2 <= tau) & (ik != i0)
                best = jnp.minimum(best, jnp.where(keep, val, jnp.inf))
            best_v[pl.ds(off, 16)] = best
            return carry

        lax.fori_loop(0, ng, group, 0)
        pltpu.sync_copy(best_v, out_hbm.at[bb, pl.ds(start, rw)])

    return sc_kernel(pts_flat, spt_flat, keys_flat)


def kernel(points, spoints):
    points_t = jnp.swapaxes(points, 1, 2)
    spoints_t = jnp.swapaxes(spoints, 1, 2)
    outs = []
    for bb in range(points.shape[0]):
        keys = _topk_call(points[bb:bb + 1], spoints_t[bb:bb + 1])
        outs.append(_voro_sc(points_t[bb:bb + 1], spoints_t[bb:bb + 1], keys))
    return jnp.concatenate(outs, axis=0)
